# speedup vs baseline: 3.2405x; 3.2405x over previous
import jax
import jax.numpy as jnp
from jax import lax
from jax.experimental import pallas as pl
from jax.experimental.pallas import tpu as pltpu

_EPS = 1e-5
_D, _H, _WD = 32, 128, 128
_M = _D * _H * _WD
_WT = _D * (_H + 2) * (_WD + 2)
_CB = 8


def _stats_kernel(x_ref, s1_ref, s2_ref, sw_ref):
    x = x_ref[...]
    cs = jnp.sum(x, axis=1)
    cs2 = jnp.sum(x * x, axis=1)
    s1row = jnp.sum(cs, axis=1)
    s1_ref[...] = s1row
    s2_ref[...] = jnp.sum(cs2, axis=1)
    sw_ref[...] = s1row + cs[:, 1, :] + cs[:, _H - 2, :]


def _finish_kernel(s1_ref, s2_ref, sw_ref, w_ref, o_ref):
    S1 = s1_ref[...]
    S2 = s2_ref[...]
    SW = sw_ref[...]
    j = lax.broadcasted_iota(jnp.int32, S1.shape, 1)
    cw = jnp.where((j == 1) | (j == _WD - 2), 2.0, 1.0)
    s1 = jnp.sum(S1, axis=1, keepdims=True)
    s2 = jnp.sum(S2, axis=1, keepdims=True)
    sw = jnp.sum(SW * cw, axis=1, keepdims=True)
    mean = s1 * (1.0 / _M)
    var = s2 * (1.0 / _M) - mean * mean
    rstd = lax.rsqrt(var + _EPS)
    desc = rstd * (sw - mean * _WT) * (1.0 / _WT)
    Wm = w_ref[...]
    C = Wm.shape[0]
    rows = []
    for n in range(o_ref.shape[0]):
        dn = desc[n * C:(n + 1) * C]
        rows.append(jnp.sum(dn * Wm, axis=0, keepdims=True))
    o_ref[...] = jnp.concatenate(rows, axis=0)


def kernel(X, W):
    N, C, D, H, Wd = X.shape
    NC = N * C
    Xr = X.reshape(NC, D, H, Wd)
    s1, s2, sw = pl.pallas_call(
        _stats_kernel,
        grid=(NC // _CB,),
        in_specs=[pl.BlockSpec((_CB, D, H, Wd), lambda i: (i, 0, 0, 0))],
        out_specs=[pl.BlockSpec((_CB, Wd), lambda i: (i, 0))] * 3,
        out_shape=[jax.ShapeDtypeStruct((NC, Wd), jnp.float32)] * 3,
        compiler_params=pltpu.CompilerParams(
            dimension_semantics=("parallel",),
            vmem_limit_bytes=56 * 1024 * 1024,
        ),
        name="instnorm_stats",
    )(Xr)
    out = pl.pallas_call(
        _finish_kernel,
        out_shape=jax.ShapeDtypeStruct((N, W.shape[1]), jnp.float32),
        name="instnorm_finish",
    )(s1, s2, sw, W)
    return out


# speedup vs baseline: 3.2660x; 1.0078x over previous
import jax
import jax.numpy as jnp
from jax import lax
from jax.experimental import pallas as pl
from jax.experimental.pallas import tpu as pltpu

_EPS = 1e-5
_D, _H, _WD = 32, 128, 128
_M = _D * _H * _WD
_WT = _D * (_H + 2) * (_WD + 2)
_CB = 8


def _fused_kernel(x_ref, w_ref, o_ref, s1_ref, s2_ref, sw_ref):
    i = pl.program_id(0)
    x = x_ref[...]
    cs = jnp.sum(x, axis=1)
    cs2 = jnp.sum(x * x, axis=1)
    s1row = jnp.sum(cs, axis=1)
    r = pl.ds(i * _CB, _CB)
    s1_ref[r, :] = s1row
    s2_ref[r, :] = jnp.sum(cs2, axis=1)
    sw_ref[r, :] = s1row + cs[:, 1, :] + cs[:, _H - 2, :]

    @pl.when(i == pl.num_programs(0) - 1)
    def _finish():
        S1 = s1_ref[...]
        S2 = s2_ref[...]
        SW = sw_ref[...]
        j = lax.broadcasted_iota(jnp.int32, S1.shape, 1)
        cw = jnp.where((j == 1) | (j == _WD - 2), 2.0, 1.0)
        s1 = jnp.sum(S1, axis=1, keepdims=True)
        s2 = jnp.sum(S2, axis=1, keepdims=True)
        sw = jnp.sum(SW * cw, axis=1, keepdims=True)
        mean = s1 * (1.0 / _M)
        var = s2 * (1.0 / _M) - mean * mean
        rstd = lax.rsqrt(var + _EPS)
        desc = rstd * (sw - mean * _WT) * (1.0 / _WT)
        Wm = w_ref[...]
        C = Wm.shape[0]
        rows = []
        for n in range(o_ref.shape[0]):
            dn = desc[n * C:(n + 1) * C]
            rows.append(jnp.sum(dn * Wm, axis=0, keepdims=True))
        o_ref[...] = jnp.concatenate(rows, axis=0)


def kernel(X, W):
    N, C, D, H, Wd = X.shape
    NC = N * C
    Xr = X.reshape(NC, D, H, Wd)
    return pl.pallas_call(
        _fused_kernel,
        grid=(NC // _CB,),
        in_specs=[
            pl.BlockSpec((_CB, D, H, Wd), lambda i: (i, 0, 0, 0)),
            pl.BlockSpec((C, W.shape[1]), lambda i: (0, 0)),
        ],
        out_specs=pl.BlockSpec((N, W.shape[1]), lambda i: (0, 0)),
        out_shape=jax.ShapeDtypeStruct((N, W.shape[1]), jnp.float32),
        scratch_shapes=[
            pltpu.VMEM((NC, Wd), jnp.float32),
            pltpu.VMEM((NC, Wd), jnp.float32),
            pltpu.VMEM((NC, Wd), jnp.float32),
        ],
        compiler_params=pltpu.CompilerParams(
            dimension_semantics=("arbitrary",),
            vmem_limit_bytes=56 * 1024 * 1024,
        ),
        name="instnorm_fused",
    )(Xr, W)
